# Initial kernel scaffold; baseline (speedup 1.0000x reference)
import functools

import jax
import jax.numpy as jnp
from jax import lax
from jax.experimental import pallas as pl
from jax.experimental.pallas import tpu as pltpu
from jax.experimental.pallas import tpu_sc as plsc

N = 10000
D = 128
E = 320000
G = 200

C = 128
SC_TILES = 16
NCH = -(-E // (C * SC_TILES))
PER_TILE = NCH * C
EP = PER_TILE * SC_TILES
NP = 10016
ZROWS = NP // SC_TILES
OROWS = N // SC_TILES


def _sc_aggregate(x1, src1, dst1, x2, src2, dst2, zeros_np):
    mesh = plsc.VectorSubcoreMesh(core_axis_name="c", subcore_axis_name="s")

    @functools.partial(
        pl.kernel,
        mesh=mesh,
        out_type=jax.ShapeDtypeStruct((2, N, D), jnp.float32),
        scratch_types=[
            pltpu.VMEM_SHARED((NP, D), jnp.float32),
            pltpu.VMEM((C,), jnp.int32),
            pltpu.VMEM((C,), jnp.int32),
            pltpu.VMEM((C, D), jnp.float32),
            pltpu.SemaphoreType.DMA,
        ],
    )
    def k(x1_h, s1_h, d1_h, x2_h, s2_h, d2_h, z_h, out_h,
          agg_sh, sbuf, dbuf, rows, sem):
        c = lax.axis_index("c")
        s = lax.axis_index("s")

        pltpu.sync_copy(z_h.at[pl.ds(s * ZROWS, ZROWS)],
                        agg_sh.at[pl.ds(s * ZROWS, ZROWS)])
        plsc.subcore_barrier()

        base = s * PER_TILE

        def run(x_h, src_h, dst_h):
            def body(j, carry):
                off = pl.multiple_of(base + j * C, 8)
                pltpu.sync_copy(src_h.at[pl.ds(off, C)], sbuf)
                pltpu.sync_copy(dst_h.at[pl.ds(off, C)], dbuf)
                pltpu.async_copy(x_h.at[sbuf], rows, sem).wait()
                pltpu.sync_copy(rows, agg_sh.at[dbuf], add=True)
                return carry
            lax.fori_loop(0, NCH, body, 0)

        @pl.when(c == 0)
        def _():
            run(x1_h, s1_h, d1_h)

        @pl.when(c == 1)
        def _():
            run(x2_h, s2_h, d2_h)

        plsc.subcore_barrier()
        pltpu.sync_copy(agg_sh.at[pl.ds(s * OROWS, OROWS)],
                        out_h.at[c, pl.ds(s * OROWS, OROWS)])

    return k(x1, src1, dst1, x2, src2, dst2, zeros_np)


BN = 1000
NB = N // BN


def _tc_head(agg, x, Wg, bg, Wrg, brg, W1g, b1g):
    def body(agg_ref, x_ref, w_ref, b_ref, wr_ref, br_ref, w1_ref, b1_ref,
             s_ref):
        j = pl.program_id(1)
        a = agg_ref[0]
        xb = x_ref[0]
        h = jnp.maximum(
            jnp.dot(a, w_ref[0], preferred_element_type=jnp.float32)
            + b_ref[0], 0.0)
        r = jnp.maximum(
            jnp.dot(xb, wr_ref[0], preferred_element_type=jnp.float32)
            + br_ref[0], 0.0)
        t = jnp.maximum(
            jnp.dot(h + r, w1_ref[0], preferred_element_type=jnp.float32)
            + b1_ref[0], 0.0)

        @pl.when(j == 0)
        def _():
            s_ref[...] = jnp.zeros_like(s_ref)

        s_ref[0] += jnp.sum(t, axis=0, keepdims=True)

    return pl.pallas_call(
        body,
        grid=(2, NB),
        in_specs=[
            pl.BlockSpec((1, BN, D), lambda g, j: (g, j, 0)),
            pl.BlockSpec((1, BN, D), lambda g, j: (g, j, 0)),
            pl.BlockSpec((1, D, D), lambda g, j: (g, 0, 0)),
            pl.BlockSpec((1, 1, D), lambda g, j: (g, 0, 0)),
            pl.BlockSpec((1, D, D), lambda g, j: (g, 0, 0)),
            pl.BlockSpec((1, 1, D), lambda g, j: (g, 0, 0)),
            pl.BlockSpec((1, D, D), lambda g, j: (g, 0, 0)),
            pl.BlockSpec((1, 1, D), lambda g, j: (g, 0, 0)),
        ],
        out_specs=pl.BlockSpec((1, 1, D), lambda g, j: (g, 0, 0)),
        out_shape=jax.ShapeDtypeStruct((2, 1, D), jnp.float32),
    )(agg, x, Wg, bg, Wrg, brg, W1g, b1g)


def kernel(node_feats_1, edge_index_1, edge_feats_1,
           node_feats_2, edge_index_2, edge_feats_2,
           W_g1, b_g1, Wr_g1, br_g1, W1_r1, b1_r1, W2_r1, b2_r1,
           W_g2, b_g2, Wr_g2, br_g2, W1_r2, b1_r2, W2_r2, b2_r2,
           Wp, bp):
    pad = EP - E
    src1 = jnp.concatenate([edge_index_1[0], jnp.zeros((pad,), jnp.int32)])
    dst1 = jnp.concatenate([edge_index_1[1], jnp.full((pad,), N, jnp.int32)])
    src2 = jnp.concatenate([edge_index_2[0], jnp.zeros((pad,), jnp.int32)])
    dst2 = jnp.concatenate([edge_index_2[1], jnp.full((pad,), N, jnp.int32)])
    zeros_np = jnp.zeros((NP, D), jnp.float32)

    agg = _sc_aggregate(node_feats_1, src1, dst1,
                        node_feats_2, src2, dst2, zeros_np)

    x = jnp.stack([node_feats_1, node_feats_2])
    Wg = jnp.stack([W_g1, W_g2])
    bg = jnp.stack([b_g1, b_g2]).reshape(2, 1, D)
    Wrg = jnp.stack([Wr_g1, Wr_g2]).reshape(2, D, D)
    brg = jnp.stack([br_g1, br_g2]).reshape(2, 1, D)
    W1g = jnp.stack([W1_r1, W1_r2])
    b1g = jnp.stack([b1_r1, b1_r2]).reshape(2, 1, D)

    s = _tc_head(agg, x, Wg, bg, Wrg, brg, W1g, b1g)

    g_vec = (s[0, 0] @ W2_r1 + N * b2_r1) + (s[1, 0] @ W2_r2 + N * b2_r2)
    out = g_vec @ Wp + bp
    return out.reshape(-1)


# speedup vs baseline: 5.3211x; 5.3211x over previous
import functools

import jax
import jax.numpy as jnp
from jax import lax
from jax.experimental import pallas as pl
from jax.experimental.pallas import tpu as pltpu
from jax.experimental.pallas import tpu_sc as plsc

N = 10000
D = 128
E = 320000
G = 200

C = 128
SC_TILES = 16
NCH = -(-E // (C * SC_TILES))
PER_TILE = NCH * C
EP = PER_TILE * SC_TILES
NP = 10112
ZROWS = NP // SC_TILES


def _sc_aggregate(x1, src1, dst1, x2, src2, dst2, zeros_np):
    mesh = plsc.VectorSubcoreMesh(core_axis_name="c", subcore_axis_name="s")

    @functools.partial(
        pl.kernel,
        mesh=mesh,
        out_type=jax.ShapeDtypeStruct((2, NP, D), jnp.float32),
        scratch_types=[
            pltpu.VMEM_SHARED((NP, D), jnp.float32),
            pltpu.VMEM((C,), jnp.int32),
            pltpu.VMEM((C,), jnp.int32),
            pltpu.VMEM((C, D), jnp.float32),
            pltpu.SemaphoreType.DMA,
        ],
    )
    def k(x1_h, s1_h, d1_h, x2_h, s2_h, d2_h, z_h, out_h,
          agg_sh, sbuf, dbuf, rows, sem):
        c = lax.axis_index("c")
        s = lax.axis_index("s")

        pltpu.sync_copy(z_h.at[pl.ds(s * ZROWS, ZROWS)],
                        agg_sh.at[pl.ds(s * ZROWS, ZROWS)])
        plsc.subcore_barrier()

        base = s * PER_TILE

        def run(x_h, src_h, dst_h):
            def body(j, carry):
                off = pl.multiple_of(base + j * C, 8)
                pltpu.sync_copy(src_h.at[pl.ds(off, C)], sbuf)
                pltpu.sync_copy(dst_h.at[pl.ds(off, C)], dbuf)
                pltpu.async_copy(x_h.at[sbuf], rows, sem).wait()
                pltpu.sync_copy(rows, agg_sh.at[dbuf], add=True)
                return carry
            lax.fori_loop(0, NCH, body, 0)

        @pl.when(c == 0)
        def _():
            run(x1_h, s1_h, d1_h)

        @pl.when(c == 1)
        def _():
            run(x2_h, s2_h, d2_h)

        plsc.subcore_barrier()
        pltpu.sync_copy(agg_sh.at[pl.ds(s * ZROWS, ZROWS)],
                        out_h.at[c, pl.ds(s * ZROWS, ZROWS)])

    return k(x1, src1, dst1, x2, src2, dst2, zeros_np)


BN = 1000
NB = N // BN


def _tc_head(agg, x, Wg, bg, Wrg, brg, W1g, b1g):
    def body(agg_ref, x_ref, w_ref, b_ref, wr_ref, br_ref, w1_ref, b1_ref,
             s_ref):
        j = pl.program_id(1)
        a = agg_ref[0]
        xb = x_ref[0]
        h = jnp.maximum(
            jnp.dot(a, w_ref[0], preferred_element_type=jnp.float32)
            + b_ref[0], 0.0)
        r = jnp.maximum(
            jnp.dot(xb, wr_ref[0], preferred_element_type=jnp.float32)
            + br_ref[0], 0.0)
        t = jnp.maximum(
            jnp.dot(h + r, w1_ref[0], preferred_element_type=jnp.float32)
            + b1_ref[0], 0.0)

        @pl.when(j == 0)
        def _():
            s_ref[...] = jnp.zeros_like(s_ref)

        s_ref[0] += jnp.sum(t, axis=0, keepdims=True)

    return pl.pallas_call(
        body,
        grid=(2, NB),
        in_specs=[
            pl.BlockSpec((1, BN, D), lambda g, j: (g, j, 0)),
            pl.BlockSpec((1, BN, D), lambda g, j: (g, j, 0)),
            pl.BlockSpec((1, D, D), lambda g, j: (g, 0, 0)),
            pl.BlockSpec((1, 1, D), lambda g, j: (g, 0, 0)),
            pl.BlockSpec((1, D, D), lambda g, j: (g, 0, 0)),
            pl.BlockSpec((1, 1, D), lambda g, j: (g, 0, 0)),
            pl.BlockSpec((1, D, D), lambda g, j: (g, 0, 0)),
            pl.BlockSpec((1, 1, D), lambda g, j: (g, 0, 0)),
        ],
        out_specs=pl.BlockSpec((1, 1, D), lambda g, j: (g, 0, 0)),
        out_shape=jax.ShapeDtypeStruct((2, 1, D), jnp.float32),
    )(agg, x, Wg, bg, Wrg, brg, W1g, b1g)


def kernel(node_feats_1, edge_index_1, edge_feats_1,
           node_feats_2, edge_index_2, edge_feats_2,
           W_g1, b_g1, Wr_g1, br_g1, W1_r1, b1_r1, W2_r1, b2_r1,
           W_g2, b_g2, Wr_g2, br_g2, W1_r2, b1_r2, W2_r2, b2_r2,
           Wp, bp):
    pad = EP - E
    src1 = jnp.concatenate([edge_index_1[0], jnp.zeros((pad,), jnp.int32)])
    dst1 = jnp.concatenate([edge_index_1[1], jnp.full((pad,), N, jnp.int32)])
    src2 = jnp.concatenate([edge_index_2[0], jnp.zeros((pad,), jnp.int32)])
    dst2 = jnp.concatenate([edge_index_2[1], jnp.full((pad,), N, jnp.int32)])
    zeros_np = jnp.zeros((NP, D), jnp.float32)

    agg = _sc_aggregate(node_feats_1, src1, dst1,
                        node_feats_2, src2, dst2, zeros_np)

    x = jnp.stack([node_feats_1, node_feats_2])
    Wg = jnp.stack([W_g1, W_g2])
    bg = jnp.stack([b_g1, b_g2]).reshape(2, 1, D)
    Wrg = jnp.stack([Wr_g1, Wr_g2]).reshape(2, D, D)
    brg = jnp.stack([br_g1, br_g2]).reshape(2, 1, D)
    W1g = jnp.stack([W1_r1, W1_r2])
    b1g = jnp.stack([b1_r1, b1_r2]).reshape(2, 1, D)

    s = _tc_head(agg, x, Wg, bg, Wrg, brg, W1g, b1g)

    g_vec = (s[0, 0] @ W2_r1 + N * b2_r1) + (s[1, 0] @ W2_r2 + N * b2_r2)
    out = g_vec @ Wp + bp
    return out.reshape(-1)
